# baseline (speedup 1.0000x reference)
import functools

import jax
import jax.numpy as jnp
from jax import lax
from jax.experimental import pallas as pl
from jax.experimental.pallas import tpu as pltpu
from jax.experimental.pallas import tpu_sc as plsc

VOCAB = 1000000
HID = 64
MAXLEN = 512
B = 1024
S = 512
S2 = S // 2
N = B * S

NC = 2
NS = 16
NW = NC * NS

CHUNK = 1024
KSUB = CHUNK // 128
PER_W = N // NW
NITER = PER_W // CHUNK
IDX_ROWS = PER_W // 128


def _sc_gather(ids2d, table):
    mesh = plsc.VectorSubcoreMesh(core_axis_name="c", subcore_axis_name="s")

    @functools.partial(
        pl.kernel,
        out_type=jax.ShapeDtypeStruct((N, HID), jnp.float32),
        mesh=mesh,
        scratch_types=[
            pltpu.VMEM((IDX_ROWS, 128), jnp.int32),
            pltpu.VMEM((CHUNK, HID), jnp.float32),
            pltpu.SemaphoreType.DMA,
        ],
        compiler_params=pltpu.CompilerParams(use_tc_tiling_on_sc=False),
    )
    def k(ids_hbm, table_hbm, out_hbm, idx_v, rows_v, sem):
        wid = lax.axis_index("s") * NC + lax.axis_index("c")
        idx_base = pl.multiple_of(wid * IDX_ROWS, IDX_ROWS)
        pltpu.sync_copy(ids_hbm.at[pl.ds(idx_base, IDX_ROWS)], idx_v)

        def body(i, _):
            base = pl.multiple_of(wid * PER_W + i * CHUNK, CHUNK)
            handles = []
            for ksub in range(KSUB):
                handles.append(pltpu.async_copy(
                    table_hbm.at[idx_v.at[i * KSUB + ksub]],
                    rows_v.at[pl.ds(ksub * 128, 128)],
                    sem,
                ))
            for h in handles:
                h.wait()
            pltpu.sync_copy(rows_v, out_hbm.at[pl.ds(base, CHUNK)])
            return ()

        lax.fori_loop(0, NITER, body, ())

    return k(ids2d, table)


def _tc_ln_body(g_ref, te_ref, to_ref, pos_ref, seg_ref, gamma_ref, beta_ref,
                o_ref):
    x = g_ref[...]
    pos = pos_ref[...]
    seg = seg_ref[...]
    x = x + pos[None, :, :]
    te = te_ref[...]
    to = to_ref[...]
    s0 = seg[0][None, None, :]
    s1 = seg[1][None, None, :]
    seg_l = jnp.where(te[:, :, None] == 1, s1, s0)
    seg_r = jnp.where(to[:, :, None] == 1, s1, s0)
    x = x + jnp.concatenate([seg_l, seg_r], axis=-1)
    x0 = x[:, :, :HID]
    x1 = x[:, :, HID:]

    def ln(v):
        m = jnp.mean(v, axis=-1, keepdims=True)
        c = v - m
        var = jnp.mean(c * c, axis=-1, keepdims=True)
        return c * lax.rsqrt(var + 1e-5)

    gamma = gamma_ref[...][None, None, :]
    beta = beta_ref[...][None, None, :]
    y0 = ln(x0) * gamma + beta
    y1 = ln(x1) * gamma + beta
    y = jnp.concatenate([y0[:, :, None, :], y1[:, :, None, :]], axis=2)
    o_ref[...] = y.reshape(y.shape[0], S, HID)


def _tc_ln(g2, te, to, pos2, seg, gamma, beta):
    Bb = 8
    grid = (B // Bb,)
    return pl.pallas_call(
        _tc_ln_body,
        grid=grid,
        in_specs=[
            pl.BlockSpec((Bb, S2, 128), lambda i: (i, 0, 0)),
            pl.BlockSpec((Bb, S2), lambda i: (i, 0)),
            pl.BlockSpec((Bb, S2), lambda i: (i, 0)),
            pl.BlockSpec((S2, 128), lambda i: (0, 0)),
            pl.BlockSpec((2, HID), lambda i: (0, 0)),
            pl.BlockSpec((HID,), lambda i: (0,)),
            pl.BlockSpec((HID,), lambda i: (0,)),
        ],
        out_specs=pl.BlockSpec((Bb, S, HID), lambda i: (i, 0, 0)),
        out_shape=jax.ShapeDtypeStruct((B, S, HID), jnp.float32),
    )(g2, te, to, pos2, seg, gamma, beta)


def kernel(input_ids, token_type_ids, token_table, pos_table, seg_table, gamma, beta):
    ids2d = input_ids.reshape(N // 128, 128)
    g = _sc_gather(ids2d, token_table)
    g2 = g.reshape(B, S2, 128)
    te = token_type_ids[:, 0::2]
    to = token_type_ids[:, 1::2]
    pos2 = pos_table.reshape(S2, 128)
    return _tc_ln(g2, te, to, pos2, seg_table, gamma, beta)


# speedup vs baseline: 1.1078x; 1.1078x over previous
import functools

import jax
import jax.numpy as jnp
from jax import lax
from jax.experimental import pallas as pl
from jax.experimental.pallas import tpu as pltpu
from jax.experimental.pallas import tpu_sc as plsc

VOCAB = 1000000
HID = 64
MAXLEN = 512
B = 1024
S = 512
N = B * S

NC = 2
NS = 16
NW = NC * NS

CHUNK = 1024
KSUB = CHUNK // 128
PER_W = N // NW
NITER = PER_W // CHUNK
IDX_ROWS = PER_W // 128

Bb = 8


def _sc_gather(ids2d, table):
    mesh = plsc.VectorSubcoreMesh(core_axis_name="c", subcore_axis_name="s")

    @functools.partial(
        pl.kernel,
        out_type=jax.ShapeDtypeStruct((N, HID), jnp.float32),
        mesh=mesh,
        scratch_types=[
            pltpu.VMEM((IDX_ROWS, 128), jnp.int32),
            pltpu.VMEM((CHUNK, HID), jnp.float32),
            pltpu.SemaphoreType.DMA,
        ],
        compiler_params=pltpu.CompilerParams(use_tc_tiling_on_sc=False),
    )
    def k(ids_hbm, table_hbm, out_hbm, idx_v, rows_v, sem):
        wid = lax.axis_index("s") * NC + lax.axis_index("c")
        idx_base = pl.multiple_of(wid * IDX_ROWS, IDX_ROWS)
        pltpu.sync_copy(ids_hbm.at[pl.ds(idx_base, IDX_ROWS)], idx_v)

        def body(i, _):
            base = pl.multiple_of(wid * PER_W + i * CHUNK, CHUNK)
            handles = []
            for ksub in range(KSUB):
                handles.append(pltpu.async_copy(
                    table_hbm.at[idx_v.at[i * KSUB + ksub]],
                    rows_v.at[pl.ds(ksub * 128, 128)],
                    sem,
                ))
            for h in handles:
                h.wait()
            pltpu.sync_copy(rows_v, out_hbm.at[pl.ds(base, CHUNK)])
            return ()

        lax.fori_loop(0, NITER, body, ())

    return k(ids2d, table)


def _tc_ln_body(g_ref, tt_ref, pos_ref, seg_ref, gamma_ref, beta_ref, o_ref):
    x = g_ref[...].reshape(Bb, S, HID)
    tt = tt_ref[...]
    pos = pos_ref[...]
    seg = seg_ref[...]
    x = x + pos[None, :, :]
    x = x + jnp.where((tt[:, :, None] == 1), seg[1][None, None, :],
                      seg[0][None, None, :])
    mean = jnp.mean(x, axis=-1, keepdims=True)
    xc = x - mean
    var = jnp.mean(xc * xc, axis=-1, keepdims=True)
    y = xc * lax.rsqrt(var + 1e-5)
    o_ref[...] = y * gamma_ref[...][None, None, :] + beta_ref[...][None, None, :]


def _tc_ln(g, tt, pos, seg, gamma, beta):
    grid = (B // Bb,)
    return pl.pallas_call(
        _tc_ln_body,
        grid=grid,
        in_specs=[
            pl.BlockSpec((Bb * S, HID), lambda i: (i, 0)),
            pl.BlockSpec((Bb, S), lambda i: (i, 0)),
            pl.BlockSpec((S, HID), lambda i: (0, 0)),
            pl.BlockSpec((2, HID), lambda i: (0, 0)),
            pl.BlockSpec((HID,), lambda i: (0,)),
            pl.BlockSpec((HID,), lambda i: (0,)),
        ],
        out_specs=pl.BlockSpec((Bb, S, HID), lambda i: (i, 0, 0)),
        out_shape=jax.ShapeDtypeStruct((B, S, HID), jnp.float32),
    )(g, tt, pos, seg, gamma, beta)


def kernel(input_ids, token_type_ids, token_table, pos_table, seg_table, gamma, beta):
    ids2d = input_ids.reshape(N // 128, 128)
    g = _sc_gather(ids2d, token_table)
    return _tc_ln(g, token_type_ids, pos_table, seg_table, gamma, beta)
